# baseline (speedup 1.0000x reference)
import functools

import jax
import jax.numpy as jnp
from jax import lax
from jax.experimental import pallas as pl
from jax.experimental.pallas import tpu as pltpu

SEP_ID = 102
W = 2048
WIN = W + 16


def _prepass_body(tok_ref, pos_ref, base8_ref, nch_ref, minoff_ref, maxoff_ref,
                  *, nrow, nblk, max_seq):
    mask = (tok_ref[...] == SEP_ID).astype(jnp.int32)
    within = mask
    shift = 1
    while shift < W:
        z = jnp.zeros((nrow, nblk, shift), jnp.int32)
        within = within + jnp.concatenate([z, within[:, :, :-shift]], axis=2)
        shift *= 2
    nsep = within[:, :, W - 1:W]
    cinc = nsep
    shift = 1
    while shift < nblk:
        z = jnp.zeros((nrow, shift, 1), jnp.int32)
        cinc = cinc + jnp.concatenate([z, cinc[:, :-shift, :]], axis=1)
        shift *= 2
    carry = cinc - nsep
    pos_ref[...] = within + carry

    base = jnp.clip(carry, 0, max_seq - WIN)
    base = base - base % 8
    pmax = jnp.minimum(carry + nsep, max_seq - 1)
    maxoff = pmax - base
    base8_ref[...] = base // 8
    nch_ref[...] = maxoff // 8 + 1
    minoff_ref[...] = jnp.clip(carry, 0, max_seq - 1) - base
    maxoff_ref[...] = maxoff


def _main_body(base8_s, nch_s, minoff_s, maxoff_s, pos_ref, x_ref, peblk_ref,
               pe_ref, out_ref, window, sem, *, max_seq):
    i = pl.program_id(0)
    base = base8_s[i] * 8
    window[pl.ds(0, 8), :] = peblk_ref[...]
    nch = nch_s[i]

    @pl.when(nch > 1)
    def _fetch_rest():
        def fetch(j, _):
            cp = pltpu.make_async_copy(
                pe_ref.at[pl.ds(pl.multiple_of(base + 8 * j, 8), 8), :],
                window.at[pl.ds(8 * j, 8), :],
                sem,
            )
            cp.start()
            cp.wait()
            return 0

        lax.fori_loop(1, nch, fetch, 0)

    off = jnp.clip(pos_ref[0], 0, max_seq - 1) - base
    x = x_ref[0]
    mo = minoff_s[i]
    out_ref[0] = x + window[pl.ds(mo, 1), :]

    def blend(d, _):
        row = window[pl.ds(d, 1), :]
        out_ref[0] = jnp.where(off == d, x + row, out_ref[0])
        return 0

    lax.fori_loop(mo + 1, maxoff_s[i] + 1, blend, 0)


def kernel(x, tokens, pe):
    B, L, D = x.shape
    max_seq = pe.shape[1]
    nblk = L // W
    nb = B * nblk

    tok3 = tokens.reshape(B, nblk, W)
    prepass = pl.pallas_call(
        functools.partial(_prepass_body, nrow=B, nblk=nblk, max_seq=max_seq),
        out_shape=(
            jax.ShapeDtypeStruct((B, nblk, W), jnp.int32),
            jax.ShapeDtypeStruct((B, nblk, 1), jnp.int32),
            jax.ShapeDtypeStruct((B, nblk, 1), jnp.int32),
            jax.ShapeDtypeStruct((B, nblk, 1), jnp.int32),
            jax.ShapeDtypeStruct((B, nblk, 1), jnp.int32),
        ),
    )
    pos, base8, nch, minoff, maxoff = prepass(tok3)

    grid_spec = pltpu.PrefetchScalarGridSpec(
        num_scalar_prefetch=4,
        grid=(nb,),
        in_specs=[
            pl.BlockSpec((1, W, 1), lambda i, *_: (i, 0, 0)),
            pl.BlockSpec((1, W, D), lambda i, *_: (i, 0, 0)),
            pl.BlockSpec((8, D), lambda i, base8, *_: (base8[i], 0)),
            pl.BlockSpec(memory_space=pltpu.MemorySpace.HBM),
        ],
        out_specs=pl.BlockSpec((1, W, D), lambda i, *_: (i, 0, 0)),
        scratch_shapes=[
            pltpu.VMEM((WIN, D), jnp.float32),
            pltpu.SemaphoreType.DMA,
        ],
    )
    main = pl.pallas_call(
        functools.partial(_main_body, max_seq=max_seq),
        grid_spec=grid_spec,
        out_shape=jax.ShapeDtypeStruct((nb, W, D), jnp.float32),
        compiler_params=pltpu.CompilerParams(
            dimension_semantics=("arbitrary",),
        ),
    )
    out = main(
        base8.reshape(nb), nch.reshape(nb), minoff.reshape(nb),
        maxoff.reshape(nb),
        pos.reshape(nb, W, 1), x.reshape(nb, W, D), pe[0], pe[0],
    )
    return out.reshape(B, L, D)


# speedup vs baseline: 1.1325x; 1.1325x over previous
import functools

import jax
import jax.numpy as jnp
from jax import lax
from jax.experimental import pallas as pl
from jax.experimental.pallas import tpu as pltpu

SEP_ID = 102
W = 1024
WIN = W + 16


def _prepass_body(tok_ref, base8_ref, nch_ref, minoff_ref, maxoff_ref,
                  nsep_ref, *, nrow, nblk, max_seq):
    mask = (tok_ref[...] == SEP_ID).astype(jnp.int32)
    nsep = jnp.sum(mask, axis=2, keepdims=True)
    cinc = nsep
    shift = 1
    while shift < nblk:
        z = jnp.zeros((nrow, shift, 1), jnp.int32)
        cinc = cinc + jnp.concatenate([z, cinc[:, :-shift, :]], axis=1)
        shift *= 2
    carry = cinc - nsep

    base = jnp.clip(carry, 0, max_seq - WIN)
    base = base - base % 8
    pmax = jnp.minimum(carry + nsep, max_seq - 1)
    maxoff = pmax - base
    base8_ref[...] = base // 8
    nch_ref[...] = maxoff // 8 + 1
    minoff_ref[...] = jnp.clip(carry, 0, max_seq - 1) - base
    maxoff_ref[...] = maxoff
    nsep_ref[...] = nsep


def _main_body(base8_s, nch_s, minoff_s, maxoff_s, nsep_s, tok_ref, x_ref,
               peblk_ref, pe_ref, out_ref, window, sem, *, max_seq):
    i = pl.program_id(0)
    base = base8_s[i] * 8
    window[pl.ds(0, 8), :] = peblk_ref[...]
    nch = nch_s[i]

    @pl.when(nch > 1)
    def _fetch_rest():
        def fetch(j, _):
            cp = pltpu.make_async_copy(
                pe_ref.at[pl.ds(pl.multiple_of(base + 8 * j, 8), 8), :],
                window.at[pl.ds(8 * j, 8), :],
                sem,
            )
            cp.start()
            cp.wait()
            return 0

        lax.fori_loop(1, nch, fetch, 0)

    mo = minoff_s[i]
    maxoff = maxoff_s[i]
    nsep = nsep_s[i]
    x = x_ref[0]

    mask_row = (tok_ref[0] == SEP_ID).astype(jnp.int32)
    within = mask_row
    shift = 1
    while shift < W:
        z = jnp.zeros((1, shift), jnp.int32)
        within = within + jnp.concatenate([z, within[:, :-shift]], axis=1)
        shift *= 2
    s = lax.broadcasted_iota(jnp.int32, (W, 1), 0)

    def bound(j, off):
        b_j = jnp.sum((within <= j).astype(jnp.int32))
        return off + (s >= b_j).astype(jnp.int32)

    off = lax.fori_loop(0, nsep, bound, jnp.full((W, 1), mo, jnp.int32))
    off = jnp.minimum(off, maxoff)

    out_ref[0] = x + window[pl.ds(mo, 1), :]

    def blend(d, _):
        row = window[pl.ds(d, 1), :]
        out_ref[0] = jnp.where(off == d, x + row, out_ref[0])
        return 0

    lax.fori_loop(mo + 1, maxoff + 1, blend, 0)


def kernel(x, tokens, pe):
    B, L, D = x.shape
    max_seq = pe.shape[1]
    nblk = L // W
    nb = B * nblk

    tok3 = tokens.reshape(B, nblk, W)
    prepass = pl.pallas_call(
        functools.partial(_prepass_body, nrow=B, nblk=nblk, max_seq=max_seq),
        out_shape=tuple(
            jax.ShapeDtypeStruct((B, nblk, 1), jnp.int32) for _ in range(5)
        ),
    )
    base8, nch, minoff, maxoff, nsep = prepass(tok3)

    grid_spec = pltpu.PrefetchScalarGridSpec(
        num_scalar_prefetch=5,
        grid=(nb,),
        in_specs=[
            pl.BlockSpec((1, 1, W), lambda i, *_: (i, 0, 0)),
            pl.BlockSpec((1, W, D), lambda i, *_: (i, 0, 0)),
            pl.BlockSpec((8, D), lambda i, base8, *_: (base8[i], 0)),
            pl.BlockSpec(memory_space=pltpu.MemorySpace.HBM),
        ],
        out_specs=pl.BlockSpec((1, W, D), lambda i, *_: (i, 0, 0)),
        scratch_shapes=[
            pltpu.VMEM((WIN, D), jnp.float32),
            pltpu.SemaphoreType.DMA,
        ],
    )
    main = pl.pallas_call(
        functools.partial(_main_body, max_seq=max_seq),
        grid_spec=grid_spec,
        out_shape=jax.ShapeDtypeStruct((nb, W, D), jnp.float32),
        compiler_params=pltpu.CompilerParams(
            dimension_semantics=("arbitrary",),
        ),
    )
    out = main(
        base8.reshape(nb), nch.reshape(nb), minoff.reshape(nb),
        maxoff.reshape(nb), nsep.reshape(nb),
        tokens.reshape(nb, 1, W), x.reshape(nb, W, D), pe[0], pe[0],
    )
    return out.reshape(B, L, D)


# speedup vs baseline: 1.1383x; 1.0052x over previous
import functools

import jax
import jax.numpy as jnp
from jax import lax
from jax.experimental import pallas as pl
from jax.experimental.pallas import tpu as pltpu

SEP_ID = 102
W = 1024
WIN = W + 16


def _prepass_body(tok_ref, base8_ref, nch_ref, minoff_ref, maxoff_ref,
                  nsep_ref, *, nrow, nblk, max_seq):
    mask = (tok_ref[...] == SEP_ID).astype(jnp.int32)
    nsep = jnp.sum(mask, axis=2, keepdims=True)
    cinc = nsep
    shift = 1
    while shift < nblk:
        z = jnp.zeros((nrow, shift, 1), jnp.int32)
        cinc = cinc + jnp.concatenate([z, cinc[:, :-shift, :]], axis=1)
        shift *= 2
    carry = cinc - nsep

    base = jnp.clip(carry, 0, max_seq - WIN)
    base = base - base % 8
    pmax = jnp.minimum(carry + nsep, max_seq - 1)
    maxoff = pmax - base
    base8_ref[...] = base // 8
    nch_ref[...] = maxoff // 8 + 1
    minoff_ref[...] = jnp.clip(carry, 0, max_seq - 1) - base
    maxoff_ref[...] = maxoff
    nsep_ref[...] = nsep


def _main_body(base8_s, nch_s, minoff_s, maxoff_s, nsep_s, tok_ref, x_ref,
               peblk_ref, pe_ref, out_ref, window, sem, *, max_seq):
    i = pl.program_id(0)
    base = base8_s[i] * 8
    window[pl.ds(0, 8), :] = peblk_ref[...]
    nch = nch_s[i]

    @pl.when(nch > 1)
    def _fetch_rest():
        def fetch(j, _):
            cp = pltpu.make_async_copy(
                pe_ref.at[pl.ds(pl.multiple_of(base + 8 * j, 8), 8), :],
                window.at[pl.ds(8 * j, 8), :],
                sem,
            )
            cp.start()
            cp.wait()
            return 0

        lax.fori_loop(1, nch, fetch, 0)

    mo = minoff_s[i]
    maxoff = maxoff_s[i]
    nsep = nsep_s[i]
    spread = maxoff - mo
    x = x_ref[0]

    @pl.when(spread == 0)
    def _uniform():
        out_ref[0] = x + window[pl.ds(mo, 1), :]

    @pl.when(spread > 0)
    def _general():
        mask_row = (tok_ref[0] == SEP_ID).astype(jnp.int32)
        within = mask_row
        shift = 1
        while shift < W:
            z = jnp.zeros((1, shift), jnp.int32)
            within = within + jnp.concatenate([z, within[:, :-shift]], axis=1)
            shift *= 2
        s = lax.broadcasted_iota(jnp.int32, (W, 1), 0)

        def bound(j, off):
            b_j = jnp.sum((within <= j).astype(jnp.int32))
            return off + (s >= b_j).astype(jnp.int32)

        off = lax.fori_loop(0, nsep, bound, jnp.full((W, 1), mo, jnp.int32))
        off = jnp.minimum(off, maxoff)

        @pl.when(spread == 1)
        def _two_rows():
            r0 = window[pl.ds(mo, 1), :]
            r1 = window[pl.ds(mo + 1, 1), :]
            out_ref[0] = x + jnp.where(off == mo, r0, r1)

        @pl.when(spread > 1)
        def _many_rows():
            out_ref[0] = x + window[pl.ds(mo, 1), :]

            def blend(d, _):
                row = window[pl.ds(d, 1), :]
                out_ref[0] = jnp.where(off == d, x + row, out_ref[0])
                return 0

            lax.fori_loop(mo + 1, maxoff + 1, blend, 0)


def kernel(x, tokens, pe):
    B, L, D = x.shape
    max_seq = pe.shape[1]
    nblk = L // W
    nb = B * nblk

    tok3 = tokens.reshape(B, nblk, W)
    prepass = pl.pallas_call(
        functools.partial(_prepass_body, nrow=B, nblk=nblk, max_seq=max_seq),
        out_shape=tuple(
            jax.ShapeDtypeStruct((B, nblk, 1), jnp.int32) for _ in range(5)
        ),
    )
    base8, nch, minoff, maxoff, nsep = prepass(tok3)

    grid_spec = pltpu.PrefetchScalarGridSpec(
        num_scalar_prefetch=5,
        grid=(nb,),
        in_specs=[
            pl.BlockSpec((1, 1, W), lambda i, *_: (i, 0, 0)),
            pl.BlockSpec((1, W, D), lambda i, *_: (i, 0, 0)),
            pl.BlockSpec((8, D), lambda i, base8, *_: (base8[i], 0)),
            pl.BlockSpec(memory_space=pltpu.MemorySpace.HBM),
        ],
        out_specs=pl.BlockSpec((1, W, D), lambda i, *_: (i, 0, 0)),
        scratch_shapes=[
            pltpu.VMEM((WIN, D), jnp.float32),
            pltpu.SemaphoreType.DMA,
        ],
    )
    main = pl.pallas_call(
        functools.partial(_main_body, max_seq=max_seq),
        grid_spec=grid_spec,
        out_shape=jax.ShapeDtypeStruct((nb, W, D), jnp.float32),
        compiler_params=pltpu.CompilerParams(
            dimension_semantics=("arbitrary",),
        ),
    )
    out = main(
        base8.reshape(nb), nch.reshape(nb), minoff.reshape(nb),
        maxoff.reshape(nb), nsep.reshape(nb),
        tokens.reshape(nb, 1, W), x.reshape(nb, W, D), pe[0], pe[0],
    )
    return out.reshape(B, L, D)


# speedup vs baseline: 1.2376x; 1.0872x over previous
import functools

import jax
import jax.numpy as jnp
from jax import lax
from jax.experimental import pallas as pl
from jax.experimental.pallas import tpu as pltpu

SEP_ID = 102
W = 1024
WIN = W + 16


def _aligned_base(carry, max_seq):
    base = jnp.clip(carry, 0, max_seq - WIN)
    return base - base % 8


def _body(tok_ref, x_ref, pe_ref, out_ref, win16, winext, sem, carry_s,
          *, max_seq, nblk, nb):
    i = pl.program_id(0)

    mask_row = (tok_ref[0] == SEP_ID).astype(jnp.int32)
    nsep = jnp.sum(mask_row)

    carry = jnp.where(i == 0, 0, carry_s[0])
    base = _aligned_base(carry, max_seq)
    pmax = jnp.minimum(carry + nsep, max_seq - 1)
    maxoff = pmax - base
    mo = jnp.clip(carry, 0, max_seq - 1) - base
    nch = maxoff // 8 + 1
    spread = maxoff - mo
    buf = i % 2

    @pl.when(i == 0)
    def _prime():
        pltpu.make_async_copy(
            pe_ref.at[pl.ds(pl.multiple_of(base, 8), 8), :],
            win16.at[pl.ds(0, 8), :], sem.at[0],
        ).start()

    carry_n = jnp.where((i + 1) % nblk == 0, 0, carry + nsep)
    base_n = _aligned_base(carry_n, max_seq)
    carry_s[0] = carry_n

    @pl.when(i + 1 < nb)
    def _prefetch_next():
        pltpu.make_async_copy(
            pe_ref.at[pl.ds(pl.multiple_of(base_n, 8), 8), :],
            win16.at[pl.ds(((i + 1) % 2) * 8, 8), :], sem.at[(i + 1) % 2],
        ).start()

    pltpu.make_async_copy(
        pe_ref.at[pl.ds(pl.multiple_of(base, 8), 8), :],
        win16.at[pl.ds(buf * 8, 8), :], sem.at[buf],
    ).wait()

    winext[pl.ds(0, 8), :] = win16[pl.ds(buf * 8, 8), :]

    @pl.when(nch > 1)
    def _fetch_rest():
        def fetch(j, _):
            cp = pltpu.make_async_copy(
                pe_ref.at[pl.ds(pl.multiple_of(base + 8 * j, 8), 8), :],
                winext.at[pl.ds(8 * j, 8), :], sem.at[buf],
            )
            cp.start()
            cp.wait()
            return 0

        lax.fori_loop(1, nch, fetch, 0)

    x = x_ref[0]

    @pl.when(spread == 0)
    def _uniform():
        out_ref[0] = x + winext[pl.ds(mo, 1), :]

    @pl.when(spread > 0)
    def _general():
        within = mask_row
        shift = 1
        while shift < W:
            z = jnp.zeros((1, shift), jnp.int32)
            within = within + jnp.concatenate([z, within[:, :-shift]], axis=1)
            shift *= 2
        s = lax.broadcasted_iota(jnp.int32, (W, 1), 0)

        def bound(j, off):
            b_j = jnp.sum((within <= j).astype(jnp.int32))
            return off + (s >= b_j).astype(jnp.int32)

        off = lax.fori_loop(0, nsep, bound, jnp.full((W, 1), mo, jnp.int32))
        off = jnp.minimum(off, maxoff)

        @pl.when(spread == 1)
        def _two_rows():
            r0 = winext[pl.ds(mo, 1), :]
            r1 = winext[pl.ds(mo + 1, 1), :]
            out_ref[0] = x + jnp.where(off == mo, r0, r1)

        @pl.when(spread > 1)
        def _many_rows():
            out_ref[0] = x + winext[pl.ds(mo, 1), :]

            def blend(d, _):
                row = winext[pl.ds(d, 1), :]
                out_ref[0] = jnp.where(off == d, x + row, out_ref[0])
                return 0

            lax.fori_loop(mo + 1, maxoff + 1, blend, 0)


def kernel(x, tokens, pe):
    B, L, D = x.shape
    max_seq = pe.shape[1]
    nblk = L // W
    nb = B * nblk

    call = pl.pallas_call(
        functools.partial(_body, max_seq=max_seq, nblk=nblk, nb=nb),
        grid=(nb,),
        in_specs=[
            pl.BlockSpec((1, 1, W), lambda i: (i, 0, 0)),
            pl.BlockSpec((1, W, D), lambda i: (i, 0, 0)),
            pl.BlockSpec(memory_space=pltpu.MemorySpace.HBM),
        ],
        out_specs=pl.BlockSpec((1, W, D), lambda i: (i, 0, 0)),
        scratch_shapes=[
            pltpu.VMEM((16, D), jnp.float32),
            pltpu.VMEM((WIN, D), jnp.float32),
            pltpu.SemaphoreType.DMA((2,)),
            pltpu.SMEM((1,), jnp.int32),
        ],
        out_shape=jax.ShapeDtypeStruct((nb, W, D), jnp.float32),
        compiler_params=pltpu.CompilerParams(
            dimension_semantics=("arbitrary",),
        ),
    )
    out = call(tokens.reshape(nb, 1, W), x.reshape(nb, W, D), pe[0])
    return out.reshape(B, L, D)


# speedup vs baseline: 1.2995x; 1.0501x over previous
import functools

import jax
import jax.numpy as jnp
from jax import lax
from jax.experimental import pallas as pl
from jax.experimental.pallas import tpu as pltpu

SEP_ID = 102
W = 1024
WIN = W + 16


def _aligned_base(carry, max_seq):
    base = jnp.clip(carry, 0, max_seq - WIN)
    return base - base % 8


def _body(tok_ref, x_ref, pe_ref, out_ref, win16, winext, sem, carry_s,
          *, max_seq, nblk, nb):
    i = pl.program_id(0)

    mask_row = (tok_ref[0] == SEP_ID).astype(jnp.int32)
    nsep = jnp.sum(mask_row)

    carry = jnp.where(i == 0, 0, carry_s[0])
    base = _aligned_base(carry, max_seq)
    pmax = jnp.minimum(carry + nsep, max_seq - 1)
    maxoff = pmax - base
    mo = jnp.clip(carry, 0, max_seq - 1) - base
    nch = maxoff // 8 + 1
    spread = maxoff - mo
    buf = i % 2

    @pl.when(i == 0)
    def _prime():
        pltpu.make_async_copy(
            pe_ref.at[pl.ds(pl.multiple_of(base, 8), 8), :],
            win16.at[pl.ds(0, 8), :], sem.at[0],
        ).start()

    carry_n = jnp.where((i + 1) % nblk == 0, 0, carry + nsep)
    base_n = _aligned_base(carry_n, max_seq)
    carry_s[0] = carry_n

    @pl.when(i + 1 < nb)
    def _prefetch_next():
        pltpu.make_async_copy(
            pe_ref.at[pl.ds(pl.multiple_of(base_n, 8), 8), :],
            win16.at[pl.ds(((i + 1) % 2) * 8, 8), :], sem.at[(i + 1) % 2],
        ).start()

    pltpu.make_async_copy(
        pe_ref.at[pl.ds(pl.multiple_of(base, 8), 8), :],
        win16.at[pl.ds(buf * 8, 8), :], sem.at[buf],
    ).wait()

    winext[pl.ds(0, 8), :] = win16[pl.ds(buf * 8, 8), :]

    @pl.when(nch > 1)
    def _fetch_rest():
        def fetch(j, _):
            cp = pltpu.make_async_copy(
                pe_ref.at[pl.ds(pl.multiple_of(base + 8 * j, 8), 8), :],
                winext.at[pl.ds(8 * j, 8), :], sem.at[buf],
            )
            cp.start()
            cp.wait()
            return 0

        lax.fori_loop(1, nch, fetch, 0)

    x = x_ref[0]

    @pl.when(spread == 0)
    def _uniform():
        out_ref[0] = x + winext[pl.ds(mo, 1), :]

    @pl.when(spread == 1)
    def _two_rows():
        iota_row = lax.broadcasted_iota(jnp.int32, (1, W), 1)
        b0 = jnp.min(jnp.where(mask_row > 0, iota_row, W))
        s = lax.broadcasted_iota(jnp.int32, (W, 1), 0)
        r0 = winext[pl.ds(mo, 1), :]
        r1 = winext[pl.ds(mo + 1, 1), :]
        out_ref[0] = x + jnp.where(s < b0, r0, r1)

    @pl.when(spread > 1)
    def _many_rows():
        within = mask_row
        shift = 1
        while shift < W:
            z = jnp.zeros((1, shift), jnp.int32)
            within = within + jnp.concatenate([z, within[:, :-shift]], axis=1)
            shift *= 2
        s = lax.broadcasted_iota(jnp.int32, (W, 1), 0)

        def bound(j, off):
            b_j = jnp.sum((within <= j).astype(jnp.int32))
            return off + (s >= b_j).astype(jnp.int32)

        off = lax.fori_loop(0, nsep, bound, jnp.full((W, 1), mo, jnp.int32))
        off = jnp.minimum(off, maxoff)

        out_ref[0] = x + winext[pl.ds(mo, 1), :]

        def blend(d, _):
            row = winext[pl.ds(d, 1), :]
            out_ref[0] = jnp.where(off == d, x + row, out_ref[0])
            return 0

        lax.fori_loop(mo + 1, maxoff + 1, blend, 0)


def kernel(x, tokens, pe):
    B, L, D = x.shape
    max_seq = pe.shape[1]
    nblk = L // W
    nb = B * nblk

    call = pl.pallas_call(
        functools.partial(_body, max_seq=max_seq, nblk=nblk, nb=nb),
        grid=(nb,),
        in_specs=[
            pl.BlockSpec((1, 1, W), lambda i: (i, 0, 0)),
            pl.BlockSpec((1, W, D), lambda i: (i, 0, 0)),
            pl.BlockSpec(memory_space=pltpu.MemorySpace.HBM),
        ],
        out_specs=pl.BlockSpec((1, W, D), lambda i: (i, 0, 0)),
        scratch_shapes=[
            pltpu.VMEM((16, D), jnp.float32),
            pltpu.VMEM((WIN, D), jnp.float32),
            pltpu.SemaphoreType.DMA((2,)),
            pltpu.SMEM((1,), jnp.int32),
        ],
        out_shape=jax.ShapeDtypeStruct((nb, W, D), jnp.float32),
        compiler_params=pltpu.CompilerParams(
            dimension_semantics=("arbitrary",),
        ),
    )
    out = call(tokens.reshape(nb, 1, W), x.reshape(nb, W, D), pe[0])
    return out.reshape(B, L, D)
